# Initial kernel scaffold; baseline (speedup 1.0000x reference)
import jax
import jax.numpy as jnp
from jax.experimental import pallas as pl


def kernel(x, edge_index, w, b, alpha):
    raise NotImplementedError("write your pallas kernel here")


# speedup vs baseline: 1.0189x; 1.0189x over previous
import jax
import jax.numpy as jnp
from jax.experimental import pallas as pl
from jax.experimental.pallas import tpu as pltpu

_TM = 1024
_TK = 4096


def _proj_kernel(x_ref, wt_ref, o_ref):
    o_ref[...] = jnp.dot(
        x_ref[...], wt_ref[...], preferred_element_type=jnp.float32
    ).astype(jnp.bfloat16)


def _agg_kernel(a_ref, xw_ref, b_ref, alpha_ref, o_ref):
    k = pl.program_id(1)

    @pl.when(k == 0)
    def _():
        o_ref[...] = jnp.zeros_like(o_ref)

    o_ref[...] += jnp.dot(
        a_ref[...], xw_ref[...], preferred_element_type=jnp.float32
    )

    @pl.when(k == pl.num_programs(1) - 1)
    def _():
        h = o_ref[...] + b_ref[...]
        o_ref[...] = jnp.where(h > 0, h, alpha_ref[0, 0] * h)


def kernel(x, edge_index, w, b, alpha):
    n, c_in = x.shape
    hdim = w.shape[0]

    src = edge_index[0].astype(jnp.int32)
    dst = edge_index[1].astype(jnp.int32)
    loop = jnp.arange(n, dtype=jnp.int32)
    src = jnp.concatenate([src, loop])
    dst = jnp.concatenate([dst, loop])
    ew = jnp.ones(src.shape[0], jnp.float32)
    deg = jnp.zeros((n,), jnp.float32).at[dst].add(ew)
    dinv = jnp.where(deg > 0, jax.lax.rsqrt(deg), 0.0)
    norm = dinv[dst] * ew * dinv[src]
    a = jnp.zeros((n, n), jnp.float32).at[dst, src].add(norm).astype(jnp.bfloat16)

    xw = pl.pallas_call(
        _proj_kernel,
        out_shape=jax.ShapeDtypeStruct((n, hdim), jnp.bfloat16),
        grid=(n // _TM,),
        in_specs=[
            pl.BlockSpec((_TM, c_in), lambda i: (i, 0)),
            pl.BlockSpec((c_in, hdim), lambda i: (0, 0)),
        ],
        out_specs=pl.BlockSpec((_TM, hdim), lambda i: (i, 0)),
        compiler_params=pltpu.CompilerParams(dimension_semantics=("parallel",)),
    )(x, w.T)

    b2 = b.reshape(1, hdim)
    alpha2 = jnp.asarray(alpha, jnp.float32).reshape(1, 1)

    out = pl.pallas_call(
        _agg_kernel,
        out_shape=jax.ShapeDtypeStruct((n, hdim), jnp.float32),
        grid=(n // _TM, n // _TK),
        in_specs=[
            pl.BlockSpec((_TM, _TK), lambda i, k: (i, k)),
            pl.BlockSpec((_TK, hdim), lambda i, k: (k, 0)),
            pl.BlockSpec((1, hdim), lambda i, k: (0, 0)),
            pl.BlockSpec(memory_space=pltpu.MemorySpace.SMEM),
        ],
        out_specs=pl.BlockSpec((_TM, hdim), lambda i, k: (i, 0)),
        compiler_params=pltpu.CompilerParams(
            dimension_semantics=("parallel", "arbitrary"),
            vmem_limit_bytes=100 * 1024 * 1024,
        ),
    )(a, xw, b2, alpha2)

    return out[:n, :hdim]


# speedup vs baseline: 1.0318x; 1.0126x over previous
import jax
import jax.numpy as jnp
from jax.experimental import pallas as pl
from jax.experimental.pallas import tpu as pltpu

_BE = 256
_TM = 1024
_LANE = 128


def _proj_kernel(x_ref, wt_ref, o_ref):
    o_ref[...] = jnp.dot(
        x_ref[...], wt_ref[...], preferred_element_type=jnp.float32
    )


def _make_agg_kernel(n, hdim, nc):
    s = _BE + 1
    p = hdim // _LANE

    def agg_kernel(tile_ref, used_ref, first_ref, last_ref,
                   dst_ref, nrm_ref, b_ref, alpha_ref, idx_hbm, xw_hbm,
                   o_ref,
                   xw_ref, g_ref, sidx_ref, copy_sem, xw_sem):
        c = pl.program_id(0)
        j = pl.program_id(1)

        def start_copy(jc, slot):
            pltpu.make_async_copy(
                idx_hbm.at[c, pl.ds(jc * _BE, _BE)],
                sidx_ref.at[slot],
                copy_sem.at[slot],
            ).start()

        @pl.when(j == 0)
        def _():
            pltpu.make_async_copy(xw_hbm, xw_ref, xw_sem).start()
            start_copy(0, 0)

        @pl.when(j + 1 < nc)
        def _():
            start_copy(j + 1, jax.lax.rem(j + 1, 2))

        slot = jax.lax.rem(j, 2)
        pltpu.make_async_copy(
            idx_hbm.at[c, pl.ds(j * _BE, _BE)],
            sidx_ref.at[slot],
            copy_sem.at[slot],
        ).wait()

        @pl.when(j == 0)
        def _():
            pltpu.make_async_copy(xw_hbm, xw_ref, xw_sem).wait()

        @pl.when(j < used_ref[c])
        def _():
            @pl.when(first_ref[c, j] == 1)
            def _():
                o_ref[...] = jnp.zeros_like(o_ref)

            for e in range(_BE):
                i4 = pl.multiple_of(sidx_ref[slot, e], p)
                slab = xw_ref[pl.ds(i4, p), :]
                g_ref[e:e + p * s:s, :] = slab

            dl = dst_ref[...].reshape(1, _BE)
            nv = nrm_ref[...].reshape(1, _BE)
            iota = jax.lax.broadcasted_iota(jnp.int32, (_TM, _BE), 0)
            onehot = jnp.where(iota == dl, nv, 0.0)

            g = jnp.concatenate(
                [g_ref[pl.ds(k * s, _BE), :] for k in range(p)], axis=1)
            o_ref[...] += jnp.dot(
                onehot, g, preferred_element_type=jnp.float32)

        @pl.when(last_ref[c, j] == 1)
        def _():
            h = o_ref[...] + b_ref[...]
            o_ref[...] = jnp.where(h > 0, h, alpha_ref[0, 0] * h)

    return agg_kernel


def kernel(x, edge_index, w, b, alpha):
    n, c_in = x.shape
    hdim = w.shape[0]
    e_cnt = edge_index.shape[1]
    nt = n // _TM
    ntc = nt // 2
    p = hdim // _LANE
    nc = (e_cnt + n // 2 + _BE - 1) // _BE + ntc

    src = edge_index[0].astype(jnp.int32)
    dst = edge_index[1].astype(jnp.int32)
    loop = jnp.arange(n, dtype=jnp.int32)
    src2 = jnp.concatenate([src, loop])
    dst2 = jnp.concatenate([dst, loop])

    deg = jnp.zeros((n,), jnp.float32).at[dst2].add(1.0)
    dinv = jax.lax.rsqrt(deg)
    norm = dinv[dst2] * dinv[src2]

    bucket = dst2 >> 10
    oh = (bucket[None, :] == jnp.arange(nt, dtype=jnp.int32)[:, None])
    ranks = jnp.cumsum(oh.astype(jnp.int32), axis=1)
    rank_e = jnp.take_along_axis(ranks, bucket[None, :], axis=0)[0] - 1
    counts = ranks[:, -1]
    nch = (counts + _BE - 1) // _BE
    nch_c = nch.reshape(2, ntc)
    bstart = jnp.cumsum(nch_c, axis=1) - nch_c
    starts_flat = (bstart.reshape(nt) * _BE).astype(jnp.int32)
    core_e = bucket // ntc
    pos = starts_flat[bucket] + rank_e
    packed = src2 * 1024 + (dst2 & 1023)
    grid_i = jnp.zeros((2, nc * _BE), jnp.int32).at[core_e, pos].set(packed)
    normg = jnp.zeros((2, nc * _BE), jnp.float32).at[core_e, pos].set(norm)
    idx4 = ((grid_i >> 10) * p).astype(jnp.int32)
    dstloc = (grid_i & 1023).reshape(2 * nc, 1, _BE)
    normv = normg.reshape(2 * nc, 1, _BE)

    used = jnp.sum(nch_c, axis=1).astype(jnp.int32)
    jj = jnp.arange(nc, dtype=jnp.int32)
    tile_local = jnp.sum(
        (jj[None, None, :] >= bstart[:, :, None]).astype(jnp.int32), axis=1) - 1
    tile_arr = (tile_local
                + jnp.array([[0], [ntc]], jnp.int32)).astype(jnp.int32)
    bend = bstart + nch_c
    first_arr = jnp.any(
        jj[None, None, :] == bstart[:, :, None], axis=1).astype(jnp.int32)
    last_arr = jnp.any(
        jj[None, None, :] == (bend - 1)[:, :, None], axis=1).astype(jnp.int32)

    xw = pl.pallas_call(
        _proj_kernel,
        out_shape=jax.ShapeDtypeStruct((n, hdim), jnp.float32),
        grid=(n // _TM,),
        in_specs=[
            pl.BlockSpec((_TM, c_in), lambda i: (i, 0)),
            pl.BlockSpec((c_in, hdim), lambda i: (0, 0)),
        ],
        out_specs=pl.BlockSpec((_TM, hdim), lambda i: (i, 0)),
        compiler_params=pltpu.CompilerParams(dimension_semantics=("parallel",)),
    )(x, w.T)
    xw_cr = xw.reshape(n * p, _LANE)

    alpha2 = jnp.asarray(alpha, jnp.float32).reshape(1, 1)
    b2 = b.reshape(1, hdim)
    s = _BE + 1

    out = pl.pallas_call(
        _make_agg_kernel(n, hdim, nc),
        out_shape=jax.ShapeDtypeStruct((n, hdim), jnp.float32),
        grid_spec=pltpu.PrefetchScalarGridSpec(
            num_scalar_prefetch=4,
            grid=(2, nc),
            in_specs=[
                pl.BlockSpec((1, 1, _BE), lambda c, j, *_: (c * nc + j, 0, 0)),
                pl.BlockSpec((1, 1, _BE), lambda c, j, *_: (c * nc + j, 0, 0)),
                pl.BlockSpec((1, hdim), lambda c, j, *_: (0, 0)),
                pl.BlockSpec(memory_space=pltpu.MemorySpace.SMEM),
                pl.BlockSpec(memory_space=pl.ANY),
                pl.BlockSpec(memory_space=pl.ANY),
            ],
            out_specs=pl.BlockSpec(
                (_TM, hdim), lambda c, j, tile, *_: (tile[c, j], 0)),
            scratch_shapes=[
                pltpu.VMEM((n * p, _LANE), jnp.float32),
                pltpu.VMEM((p * s, _LANE), jnp.float32),
                pltpu.SMEM((2, _BE), jnp.int32),
                pltpu.SemaphoreType.DMA((2,)),
                pltpu.SemaphoreType.DMA,
            ],
        ),
        compiler_params=pltpu.CompilerParams(
            dimension_semantics=("parallel", "arbitrary"),
            vmem_limit_bytes=56 * 1024 * 1024,
        ),
    )(tile_arr, used, first_arr, last_arr,
      dstloc, normv, b2, alpha2, idx4, xw_cr)

    return out


# speedup vs baseline: 2.8170x; 2.7302x over previous
import jax
import jax.numpy as jnp
from jax.experimental import pallas as pl
from jax.experimental.pallas import tpu as pltpu

_BE = 256
_TM = 1024
_LANE = 128


def _proj_kernel(x_ref, wt_ref, dinv_ref, o_ref):
    o_ref[...] = dinv_ref[...] * jnp.dot(
        x_ref[...], wt_ref[...], preferred_element_type=jnp.float32
    )


def _make_deg_kernel(nc):
    def deg_kernel(tile_ref, used_ref, first_ref, last_ref,
                   dst_ref, o_ref):
        c = pl.program_id(0)
        j = pl.program_id(1)

        @pl.when(j < used_ref[c])
        def _():
            @pl.when(first_ref[c, j] == 1)
            def _():
                o_ref[...] = jnp.zeros_like(o_ref)

            dl = dst_ref[...].reshape(1, _BE)
            iota = jax.lax.broadcasted_iota(jnp.int32, (_TM, _BE), 0)
            o_ref[...] += jnp.sum(
                jnp.where(iota == dl, 1.0, 0.0), axis=1, keepdims=True)

    return deg_kernel


def _make_agg_kernel(n, hdim, nc):
    s = _BE + 1
    p = hdim // _LANE

    def agg_kernel(tile_ref, used_ref, first_ref, last_ref,
                   dst_ref, b_ref, alpha_ref, dinv_ref, xwt_ref,
                   idx_hbm, xw_hbm,
                   o_ref,
                   xw_ref, g_ref, sidx_ref, copy_sem, xw_sem):
        c = pl.program_id(0)
        j = pl.program_id(1)

        def start_copy(jc, slot):
            pltpu.make_async_copy(
                idx_hbm.at[c, pl.ds(jc * _BE, _BE)],
                sidx_ref.at[slot],
                copy_sem.at[slot],
            ).start()

        @pl.when(j == 0)
        def _():
            pltpu.make_async_copy(xw_hbm, xw_ref, xw_sem).start()
            start_copy(0, 0)

        @pl.when(j + 1 < nc)
        def _():
            start_copy(j + 1, jax.lax.rem(j + 1, 2))

        slot = jax.lax.rem(j, 2)
        pltpu.make_async_copy(
            idx_hbm.at[c, pl.ds(j * _BE, _BE)],
            sidx_ref.at[slot],
            copy_sem.at[slot],
        ).wait()

        @pl.when(j == 0)
        def _():
            pltpu.make_async_copy(xw_hbm, xw_ref, xw_sem).wait()

        @pl.when(j < used_ref[c])
        def _():
            @pl.when(first_ref[c, j] == 1)
            def _():
                o_ref[...] = jnp.zeros_like(o_ref)

            for e in range(_BE):
                i4 = pl.multiple_of(sidx_ref[slot, e], p)
                slab = xw_ref[pl.ds(i4, p), :]
                g_ref[e:e + p * s:s, :] = slab

            dl = dst_ref[...].reshape(1, _BE)
            iota = jax.lax.broadcasted_iota(jnp.int32, (_TM, _BE), 0)
            onehot = jnp.where(iota == dl, 1.0, 0.0)

            g = jnp.concatenate(
                [g_ref[pl.ds(k * s, _BE), :] for k in range(p)], axis=1)
            o_ref[...] += jnp.dot(
                onehot, g, preferred_element_type=jnp.float32)

        @pl.when(last_ref[c, j] == 1)
        def _():
            h = dinv_ref[...] * (o_ref[...] + xwt_ref[...]) + b_ref[...]
            o_ref[...] = jnp.where(h > 0, h, alpha_ref[0, 0] * h)

    return agg_kernel


def kernel(x, edge_index, w, b, alpha):
    n, c_in = x.shape
    hdim = w.shape[0]
    e_cnt = edge_index.shape[1]
    nt = n // _TM
    ntc = nt // 2
    p = hdim // _LANE
    nc = (e_cnt + _BE - 1) // _BE + ntc

    src = edge_index[0].astype(jnp.int32)
    dst = edge_index[1].astype(jnp.int32)

    bucket = dst >> 10
    oh = (bucket[None, :] == jnp.arange(nt, dtype=jnp.int32)[:, None])
    ohi = oh.astype(jnp.int32)
    ranks = jnp.cumsum(ohi, axis=1)
    rank_e = jnp.sum(ohi * ranks, axis=0) - 1
    counts = ranks[:, -1]
    nch = jnp.maximum((counts + _BE - 1) // _BE, 1)
    nch_c = nch.reshape(2, ntc)
    bstart = jnp.cumsum(nch_c, axis=1) - nch_c
    starts_flat = (bstart.reshape(nt) * _BE).astype(jnp.int32)
    start_e = jnp.sum(ohi * starts_flat[:, None], axis=0)
    core_e = bucket // ntc
    pos = start_e + rank_e
    packed = src * 1024 + (dst & 1023)
    grid_i = jnp.full((2, nc * _BE), -1, jnp.int32).at[core_e, pos].set(packed)
    pad = grid_i < 0
    idx4 = jnp.where(pad, 0, (grid_i >> 10) * p).astype(jnp.int32)
    dstloc = jnp.where(pad, 2048, grid_i & 1023).reshape(2 * nc, 1, _BE)

    used = jnp.sum(nch_c, axis=1).astype(jnp.int32)
    jj = jnp.arange(nc, dtype=jnp.int32)
    tile_local = jnp.sum(
        (jj[None, None, :] >= bstart[:, :, None]).astype(jnp.int32), axis=1) - 1
    tile_arr = (tile_local
                + jnp.array([[0], [ntc]], jnp.int32)).astype(jnp.int32)
    bend = bstart + nch_c
    first_arr = jnp.any(
        jj[None, None, :] == bstart[:, :, None], axis=1).astype(jnp.int32)
    last_arr = jnp.any(
        jj[None, None, :] == (bend - 1)[:, :, None], axis=1).astype(jnp.int32)

    deg = pl.pallas_call(
        _make_deg_kernel(nc),
        out_shape=jax.ShapeDtypeStruct((n, 1), jnp.float32),
        grid_spec=pltpu.PrefetchScalarGridSpec(
            num_scalar_prefetch=4,
            grid=(2, nc),
            in_specs=[
                pl.BlockSpec((1, 1, _BE), lambda c, j, *_: (c * nc + j, 0, 0)),
            ],
            out_specs=pl.BlockSpec(
                (_TM, 1), lambda c, j, tile, *_: (tile[c, j], 0)),
        ),
        compiler_params=pltpu.CompilerParams(
            dimension_semantics=("parallel", "arbitrary")),
    )(tile_arr, used, first_arr, last_arr, dstloc)

    dinv = jax.lax.rsqrt(deg + 1.0)

    xwp = pl.pallas_call(
        _proj_kernel,
        out_shape=jax.ShapeDtypeStruct((n, hdim), jnp.float32),
        grid=(n // _TM,),
        in_specs=[
            pl.BlockSpec((_TM, c_in), lambda i: (i, 0)),
            pl.BlockSpec((c_in, hdim), lambda i: (0, 0)),
            pl.BlockSpec((_TM, 1), lambda i: (i, 0)),
        ],
        out_specs=pl.BlockSpec((_TM, hdim), lambda i: (i, 0)),
        compiler_params=pltpu.CompilerParams(dimension_semantics=("parallel",)),
    )(x, w.T, dinv)
    xw_cr = xwp.reshape(n * p, _LANE)

    alpha2 = jnp.asarray(alpha, jnp.float32).reshape(1, 1)
    b2 = b.reshape(1, hdim)
    s = _BE + 1

    out = pl.pallas_call(
        _make_agg_kernel(n, hdim, nc),
        out_shape=jax.ShapeDtypeStruct((n, hdim), jnp.float32),
        grid_spec=pltpu.PrefetchScalarGridSpec(
            num_scalar_prefetch=4,
            grid=(2, nc),
            in_specs=[
                pl.BlockSpec((1, 1, _BE), lambda c, j, *_: (c * nc + j, 0, 0)),
                pl.BlockSpec((1, hdim), lambda c, j, *_: (0, 0)),
                pl.BlockSpec(memory_space=pltpu.MemorySpace.SMEM),
                pl.BlockSpec((_TM, 1), lambda c, j, tile, *_: (tile[c, j], 0)),
                pl.BlockSpec(
                    (_TM, hdim), lambda c, j, tile, *_: (tile[c, j], 0)),
                pl.BlockSpec(memory_space=pl.ANY),
                pl.BlockSpec(memory_space=pl.ANY),
            ],
            out_specs=pl.BlockSpec(
                (_TM, hdim), lambda c, j, tile, *_: (tile[c, j], 0)),
            scratch_shapes=[
                pltpu.VMEM((n * p, _LANE), jnp.float32),
                pltpu.VMEM((p * s, _LANE), jnp.float32),
                pltpu.SMEM((2, _BE), jnp.int32),
                pltpu.SemaphoreType.DMA((2,)),
                pltpu.SemaphoreType.DMA,
            ],
        ),
        compiler_params=pltpu.CompilerParams(
            dimension_semantics=("parallel", "arbitrary"),
            vmem_limit_bytes=56 * 1024 * 1024,
        ),
    )(tile_arr, used, first_arr, last_arr,
      dstloc, b2, alpha2, dinv, xwp, idx4, xw_cr)

    return out


# speedup vs baseline: 4.4924x; 1.5947x over previous
import jax
import jax.numpy as jnp
from jax.experimental import pallas as pl
from jax.experimental.pallas import tpu as pltpu

_BE = 1024
_TM = 1024
_LANE = 128


def _proj_kernel(x_ref, wt_ref, dinv_ref, o_ref):
    o_ref[...] = dinv_ref[...] * jnp.dot(
        x_ref[...], wt_ref[...], preferred_element_type=jnp.float32
    )


def _make_deg_kernel(nc):
    def deg_kernel(tile_ref, used_ref, first_ref, last_ref,
                   dst_ref, o_ref):
        c = pl.program_id(0)
        j = pl.program_id(1)

        @pl.when(j < used_ref[c])
        def _():
            @pl.when(first_ref[c, j] == 1)
            def _():
                o_ref[...] = jnp.zeros_like(o_ref)

            dl = dst_ref[...].reshape(1, _BE)
            iota = jax.lax.broadcasted_iota(jnp.int32, (_TM, _BE), 0)
            o_ref[...] += jnp.sum(
                jnp.where(iota == dl, 1.0, 0.0), axis=1, keepdims=True)

    return deg_kernel


def _make_agg_kernel(n, hdim, nc):
    s = _BE + 1
    p = hdim // _LANE

    def agg_kernel(tile_ref, used_ref, first_ref, last_ref,
                   dst_ref, b_ref, alpha_ref, dinv_ref, xwt_ref,
                   idx_hbm, xw_hbm,
                   o_ref,
                   xw_ref, g_ref, sidx_ref, copy_sem, xw_sem):
        c = pl.program_id(0)
        j = pl.program_id(1)

        def start_copy(jc, slot):
            pltpu.make_async_copy(
                idx_hbm.at[c, pl.ds(jc * _BE, _BE)],
                sidx_ref.at[slot],
                copy_sem.at[slot],
            ).start()

        @pl.when(j == 0)
        def _():
            pltpu.make_async_copy(xw_hbm, xw_ref, xw_sem).start()
            start_copy(0, 0)

        @pl.when(j + 1 < nc)
        def _():
            start_copy(j + 1, jax.lax.rem(j + 1, 2))

        slot = jax.lax.rem(j, 2)
        pltpu.make_async_copy(
            idx_hbm.at[c, pl.ds(j * _BE, _BE)],
            sidx_ref.at[slot],
            copy_sem.at[slot],
        ).wait()

        @pl.when(j == 0)
        def _():
            pltpu.make_async_copy(xw_hbm, xw_ref, xw_sem).wait()

        @pl.when(j < used_ref[c])
        def _():
            @pl.when(first_ref[c, j] == 1)
            def _():
                o_ref[...] = jnp.zeros_like(o_ref)

            for e in range(_BE):
                i4 = pl.multiple_of(sidx_ref[slot, e], p)
                slab = xw_ref[pl.ds(i4, p), :]
                g_ref[e:e + p * s:s, :] = slab

            dl = dst_ref[...].reshape(1, _BE)
            iota = jax.lax.broadcasted_iota(jnp.int32, (_TM, _BE), 0)
            onehot = jnp.where(iota == dl, 1.0, 0.0)

            g = jnp.concatenate(
                [g_ref[pl.ds(k * s, _BE), :] for k in range(p)], axis=1)
            o_ref[...] += jnp.dot(
                onehot, g, preferred_element_type=jnp.float32)

        @pl.when(last_ref[c, j] == 1)
        def _():
            h = dinv_ref[...] * (o_ref[...] + xwt_ref[...]) + b_ref[...]
            o_ref[...] = jnp.where(h > 0, h, alpha_ref[0, 0] * h)

    return agg_kernel


def kernel(x, edge_index, w, b, alpha):
    n, c_in = x.shape
    hdim = w.shape[0]
    e_cnt = edge_index.shape[1]
    nt = n // _TM
    ntc = nt // 2
    p = hdim // _LANE
    nc = (e_cnt + _BE - 1) // _BE + ntc

    src = edge_index[0].astype(jnp.int32)
    dst = edge_index[1].astype(jnp.int32)

    bucket = dst >> 10
    oh = (bucket[None, :] == jnp.arange(nt, dtype=jnp.int32)[:, None])
    ohi = oh.astype(jnp.int32)
    ranks = jnp.cumsum(ohi, axis=1)
    rank_e = jnp.sum(ohi * ranks, axis=0) - 1
    counts = ranks[:, -1]
    nch = jnp.maximum((counts + _BE - 1) // _BE, 1)
    nch_c = nch.reshape(2, ntc)
    bstart = jnp.cumsum(nch_c, axis=1) - nch_c
    starts_flat = (bstart.reshape(nt) * _BE).astype(jnp.int32)
    start_e = jnp.sum(ohi * starts_flat[:, None], axis=0)
    core_e = bucket // ntc
    pos = start_e + rank_e
    packed = src * 1024 + (dst & 1023)
    grid_i = jnp.full((2, nc * _BE), -1, jnp.int32).at[core_e, pos].set(packed)
    pad = grid_i < 0
    idx4 = jnp.where(pad, 0, (grid_i >> 10) * p).astype(jnp.int32)
    dstloc = jnp.where(pad, 2048, grid_i & 1023).reshape(2 * nc, 1, _BE)

    used = jnp.sum(nch_c, axis=1).astype(jnp.int32)
    jj = jnp.arange(nc, dtype=jnp.int32)
    tile_local = jnp.sum(
        (jj[None, None, :] >= bstart[:, :, None]).astype(jnp.int32), axis=1) - 1
    tile_arr = (tile_local
                + jnp.array([[0], [ntc]], jnp.int32)).astype(jnp.int32)
    bend = bstart + nch_c
    first_arr = jnp.any(
        jj[None, None, :] == bstart[:, :, None], axis=1).astype(jnp.int32)
    last_arr = jnp.any(
        jj[None, None, :] == (bend - 1)[:, :, None], axis=1).astype(jnp.int32)

    deg = pl.pallas_call(
        _make_deg_kernel(nc),
        out_shape=jax.ShapeDtypeStruct((n, 1), jnp.float32),
        grid_spec=pltpu.PrefetchScalarGridSpec(
            num_scalar_prefetch=4,
            grid=(2, nc),
            in_specs=[
                pl.BlockSpec((1, 1, _BE), lambda c, j, *_: (c * nc + j, 0, 0)),
            ],
            out_specs=pl.BlockSpec(
                (_TM, 1), lambda c, j, tile, *_: (tile[c, j], 0)),
        ),
        compiler_params=pltpu.CompilerParams(
            dimension_semantics=("parallel", "arbitrary")),
    )(tile_arr, used, first_arr, last_arr, dstloc)

    dinv = jax.lax.rsqrt(deg + 1.0)

    xwp = pl.pallas_call(
        _proj_kernel,
        out_shape=jax.ShapeDtypeStruct((n, hdim), jnp.float32),
        grid=(n // _TM,),
        in_specs=[
            pl.BlockSpec((_TM, c_in), lambda i: (i, 0)),
            pl.BlockSpec((c_in, hdim), lambda i: (0, 0)),
            pl.BlockSpec((_TM, 1), lambda i: (i, 0)),
        ],
        out_specs=pl.BlockSpec((_TM, hdim), lambda i: (i, 0)),
        compiler_params=pltpu.CompilerParams(dimension_semantics=("parallel",)),
    )(x, w.T, dinv)
    xw_cr = xwp.reshape(n * p, _LANE)

    alpha2 = jnp.asarray(alpha, jnp.float32).reshape(1, 1)
    b2 = b.reshape(1, hdim)
    s = _BE + 1

    out = pl.pallas_call(
        _make_agg_kernel(n, hdim, nc),
        out_shape=jax.ShapeDtypeStruct((n, hdim), jnp.float32),
        grid_spec=pltpu.PrefetchScalarGridSpec(
            num_scalar_prefetch=4,
            grid=(2, nc),
            in_specs=[
                pl.BlockSpec((1, 1, _BE), lambda c, j, *_: (c * nc + j, 0, 0)),
                pl.BlockSpec((1, hdim), lambda c, j, *_: (0, 0)),
                pl.BlockSpec(memory_space=pltpu.MemorySpace.SMEM),
                pl.BlockSpec((_TM, 1), lambda c, j, tile, *_: (tile[c, j], 0)),
                pl.BlockSpec(
                    (_TM, hdim), lambda c, j, tile, *_: (tile[c, j], 0)),
                pl.BlockSpec(memory_space=pl.ANY),
                pl.BlockSpec(memory_space=pl.ANY),
            ],
            out_specs=pl.BlockSpec(
                (_TM, hdim), lambda c, j, tile, *_: (tile[c, j], 0)),
            scratch_shapes=[
                pltpu.VMEM((n * p, _LANE), jnp.float32),
                pltpu.VMEM((p * s, _LANE), jnp.float32),
                pltpu.SMEM((2, _BE), jnp.int32),
                pltpu.SemaphoreType.DMA((2,)),
                pltpu.SemaphoreType.DMA,
            ],
        ),
        compiler_params=pltpu.CompilerParams(
            dimension_semantics=("parallel", "arbitrary"),
            vmem_limit_bytes=56 * 1024 * 1024,
        ),
    )(tile_arr, used, first_arr, last_arr,
      dstloc, b2, alpha2, dinv, xwp, idx4, xw_cr)

    return out


# speedup vs baseline: 7.5652x; 1.6840x over previous
import jax
import jax.numpy as jnp
from jax.experimental import pallas as pl
from jax.experimental.pallas import tpu as pltpu

_BE = 1024
_TM = 1024
_LANE = 128


def _proj_kernel(x_ref, wt_ref, dinv_ref, o_ref):
    o_ref[...] = dinv_ref[...] * jnp.dot(
        x_ref[...], wt_ref[...], preferred_element_type=jnp.float32
    )


def _make_deg_kernel(nc):
    def deg_kernel(tile_ref, used_ref, first_ref, last_ref, cidx_ref,
                   dst_ref, o_ref):
        c = pl.program_id(0)
        j = pl.program_id(1)

        @pl.when(j < used_ref[c])
        def _():
            @pl.when(first_ref[c, j] == 1)
            def _():
                o_ref[...] = jnp.zeros_like(o_ref)

            dl = dst_ref[...].reshape(1, _BE) - tile_ref[c, j] * _TM
            iota = jax.lax.broadcasted_iota(jnp.int32, (_TM, _BE), 0)
            o_ref[...] += jnp.sum(
                jnp.where(iota == dl, 1.0, 0.0), axis=1, keepdims=True)

    return deg_kernel


def _make_agg_kernel(n, hdim, nc):
    s = _BE + 1
    p = hdim // _LANE

    def agg_kernel(tile_ref, used_ref, first_ref, last_ref, cidx_ref,
                   dst_ref, b_ref, alpha_ref, dinv_ref, xwt_ref,
                   idx_hbm, xw_hbm,
                   o_ref,
                   xw_ref, g_ref, sidx_ref, copy_sem, xw_sem):
        c = pl.program_id(0)
        j = pl.program_id(1)

        def start_copy(jc, slot):
            pltpu.make_async_copy(
                idx_hbm.at[pl.ds(cidx_ref[c, jc] * _BE, _BE)],
                sidx_ref.at[slot],
                copy_sem.at[slot],
            ).start()

        @pl.when(j == 0)
        def _():
            pltpu.make_async_copy(xw_hbm, xw_ref, xw_sem).start()
            start_copy(0, 0)

        @pl.when(j + 1 < nc)
        def _():
            start_copy(j + 1, jax.lax.rem(j + 1, 2))

        slot = jax.lax.rem(j, 2)
        pltpu.make_async_copy(
            idx_hbm.at[pl.ds(cidx_ref[c, j] * _BE, _BE)],
            sidx_ref.at[slot],
            copy_sem.at[slot],
        ).wait()

        @pl.when(j == 0)
        def _():
            pltpu.make_async_copy(xw_hbm, xw_ref, xw_sem).wait()

        @pl.when(j < used_ref[c])
        def _():
            @pl.when(first_ref[c, j] == 1)
            def _():
                o_ref[...] = jnp.zeros_like(o_ref)

            for e in range(_BE):
                i4 = pl.multiple_of(sidx_ref[slot, e], p)
                slab = xw_ref[pl.ds(i4, p), :]
                g_ref[e:e + p * s:s, :] = slab

            dl = dst_ref[...].reshape(1, _BE) - tile_ref[c, j] * _TM
            iota = jax.lax.broadcasted_iota(jnp.int32, (_TM, _BE), 0)
            onehot = jnp.where(iota == dl, 1.0, 0.0)

            g = jnp.concatenate(
                [g_ref[pl.ds(k * s, _BE), :] for k in range(p)], axis=1)
            o_ref[...] += jnp.dot(
                onehot, g, preferred_element_type=jnp.float32)

        @pl.when(last_ref[c, j] == 1)
        def _():
            h = dinv_ref[...] * (o_ref[...] + xwt_ref[...]) + b_ref[...]
            o_ref[...] = jnp.where(h > 0, h, alpha_ref[0, 0] * h)

    return agg_kernel


def kernel(x, edge_index, w, b, alpha):
    n, c_in = x.shape
    hdim = w.shape[0]
    e_cnt = edge_index.shape[1]
    nt = n // _TM
    ntc = nt // 2
    p = hdim // _LANE
    nch = (e_cnt + _BE - 1) // _BE
    nc = nch + ntc

    src = edge_index[0].astype(jnp.int32)
    dst = edge_index[1].astype(jnp.int32)

    ds, ss = jax.lax.sort_key_val(dst, src)
    padlen = nch * _BE - e_cnt
    if padlen:
        ds = jnp.concatenate([ds, jnp.full((padlen,), n, jnp.int32)])
        ss = jnp.concatenate([ss, jnp.zeros((padlen,), jnp.int32)])
    idx4 = ss * p
    dst3 = ds.reshape(nch, 1, _BE)

    t16 = jnp.arange(nt, dtype=jnp.int32)
    cnt = jnp.sum(
        ((dst[None, :] >> 10) == t16[:, None]).astype(jnp.int32), axis=1)
    cstart = (jnp.cumsum(cnt) - cnt).reshape(2, ntc)
    cend = cstart + cnt.reshape(2, ntc)
    empty = cend == cstart
    klo = jnp.where(empty, 0, cstart // _BE)
    khi = jnp.where(empty, 0, (cend - 1) // _BE)
    nslots = khi - klo + 1
    soff = jnp.cumsum(nslots, axis=1) - nslots
    used = jnp.sum(nslots, axis=1).astype(jnp.int32)

    ssl = jnp.arange(nc, dtype=jnp.int32)
    ge = (ssl[None, None, :] >= soff[:, :, None])
    tl = jnp.sum(ge.astype(jnp.int32), axis=1) - 1
    tl1 = (tl[:, None, :]
           == jnp.arange(ntc, dtype=jnp.int32)[None, :, None])

    def pick(a):
        return jnp.sum(jnp.where(tl1, a[:, :, None], 0), axis=1)

    klo_s = pick(klo)
    soff_s = pick(soff)
    nsl_s = pick(nslots)
    valid = ssl[None, :] < used[:, None]
    cidx = jnp.where(valid, klo_s + (ssl[None, :] - soff_s), 0)
    cidx = jnp.clip(cidx, 0, nch - 1).astype(jnp.int32)
    tile_arr = (tl + jnp.array([[0], [ntc]], jnp.int32)).astype(jnp.int32)
    first_arr = (ssl[None, :] == soff_s).astype(jnp.int32)
    last_arr = ((ssl[None, :] == soff_s + nsl_s - 1) & valid).astype(jnp.int32)

    deg = pl.pallas_call(
        _make_deg_kernel(nc),
        out_shape=jax.ShapeDtypeStruct((n, 1), jnp.float32),
        grid_spec=pltpu.PrefetchScalarGridSpec(
            num_scalar_prefetch=5,
            grid=(2, nc),
            in_specs=[
                pl.BlockSpec(
                    (1, 1, _BE),
                    lambda c, j, tile, u, f, l, ci: (ci[c, j], 0, 0)),
            ],
            out_specs=pl.BlockSpec(
                (_TM, 1), lambda c, j, tile, *_: (tile[c, j], 0)),
        ),
        compiler_params=pltpu.CompilerParams(
            dimension_semantics=("parallel", "arbitrary")),
    )(tile_arr, used, first_arr, last_arr, cidx, dst3)

    dinv = jax.lax.rsqrt(deg + 1.0)

    xwp = pl.pallas_call(
        _proj_kernel,
        out_shape=jax.ShapeDtypeStruct((n, hdim), jnp.float32),
        grid=(n // _TM,),
        in_specs=[
            pl.BlockSpec((_TM, c_in), lambda i: (i, 0)),
            pl.BlockSpec((c_in, hdim), lambda i: (0, 0)),
            pl.BlockSpec((_TM, 1), lambda i: (i, 0)),
        ],
        out_specs=pl.BlockSpec((_TM, hdim), lambda i: (i, 0)),
        compiler_params=pltpu.CompilerParams(dimension_semantics=("parallel",)),
    )(x, w.T, dinv)
    xw_cr = xwp.reshape(n * p, _LANE)

    alpha2 = jnp.asarray(alpha, jnp.float32).reshape(1, 1)
    b2 = b.reshape(1, hdim)
    s = _BE + 1

    out = pl.pallas_call(
        _make_agg_kernel(n, hdim, nc),
        out_shape=jax.ShapeDtypeStruct((n, hdim), jnp.float32),
        grid_spec=pltpu.PrefetchScalarGridSpec(
            num_scalar_prefetch=5,
            grid=(2, nc),
            in_specs=[
                pl.BlockSpec(
                    (1, 1, _BE),
                    lambda c, j, tile, u, f, l, ci: (ci[c, j], 0, 0)),
                pl.BlockSpec((1, hdim), lambda c, j, *_: (0, 0)),
                pl.BlockSpec(memory_space=pltpu.MemorySpace.SMEM),
                pl.BlockSpec((_TM, 1), lambda c, j, tile, *_: (tile[c, j], 0)),
                pl.BlockSpec(
                    (_TM, hdim), lambda c, j, tile, *_: (tile[c, j], 0)),
                pl.BlockSpec(memory_space=pl.ANY),
                pl.BlockSpec(memory_space=pl.ANY),
            ],
            out_specs=pl.BlockSpec(
                (_TM, hdim), lambda c, j, tile, *_: (tile[c, j], 0)),
            scratch_shapes=[
                pltpu.VMEM((n * p, _LANE), jnp.float32),
                pltpu.VMEM((p * s, _LANE), jnp.float32),
                pltpu.SMEM((2, _BE), jnp.int32),
                pltpu.SemaphoreType.DMA((2,)),
                pltpu.SemaphoreType.DMA,
            ],
        ),
        compiler_params=pltpu.CompilerParams(
            dimension_semantics=("parallel", "arbitrary"),
            vmem_limit_bytes=56 * 1024 * 1024,
        ),
    )(tile_arr, used, first_arr, last_arr, cidx,
      dst3, b2, alpha2, dinv, xwp, idx4, xw_cr)

    return out
